# baseline (speedup 1.0000x reference)
import functools

import jax
import jax.numpy as jnp
from jax import lax
from jax.experimental import pallas as pl
from jax.experimental.pallas import tpu as pltpu
from jax.experimental.pallas import tpu_sc as plsc

NC = 2
NS = 16
L = 16
NW = NC * NS


@functools.partial(jax.jit, static_argnames=("bt_per_w",))
def _sc_gather_t(params, mi_flat, *, bt_per_w):
    rows, feats = params.shape
    k = mi_flat.shape[0]
    kv = k // L
    bl = 128
    mesh = plsc.VectorSubcoreMesh(
        core_axis_name="c", subcore_axis_name="s", num_cores=NC, num_subcores=NS
    )

    def body(p_hbm, mi_hbm, out_hbm, mi_v, idx_vs, in_vs, out_vs, sems):
        sem_i0, sem_i1, sem_o0, sem_o1 = sems
        sem_i = (sem_i0, sem_i1)
        sem_o = (sem_o0, sem_o1)
        wid = lax.axis_index("s") * NC + lax.axis_index("c")
        pltpu.sync_copy(mi_hbm, mi_v)
        lane = lax.iota(jnp.int32, L)
        zvec = jnp.zeros((L,), jnp.int32)
        colbase = [(lane + g * L) * bl for g in range(kv)]
        rbase = [(lane + g * L) // 2 * 256 + (lane + g * L) % 2
                 for g in range(kv)]

        def make_in(bt_local, buf):
            bt = (wid * bt_per_w + bt_local) * bl
            return pltpu.make_async_copy(
                p_hbm.at[pl.ds(bt, bl)], in_vs.at[buf], sem_i[buf])

        def make_out(bt_local, buf):
            return pltpu.make_async_copy(
                out_vs.at[buf], out_hbm.at[idx_vs.at[buf]], sem_o[buf])

        make_in(0, 0).start()
        for bt_local in range(bt_per_w):
            buf = bt_local % 2
            if bt_local + 1 < bt_per_w:
                make_in(bt_local + 1, 1 - buf).start()
            if bt_local >= 2:
                make_out(bt_local - 2, buf).wait()
            bt2 = (wid * bt_per_w + bt_local) * 2
            for g in range(kv):
                idx_vs[buf, pl.ds(g * L, L)] = rbase[g] + bt2
            make_in(bt_local, buf).wait()
            in_v = in_vs.at[buf]
            out_v = out_vs.at[buf]
            gidx0 = tuple(mi_v[pl.ds(g * L, L)] for g in range(kv))
            sidx0 = tuple(colbase[g] for g in range(kv))

            def row_body(b, carry):
                gidx, sidx = carry
                vals = [plsc.load_gather(in_v, [zvec, gidx[g]])
                        for g in range(kv)]
                for g in range(kv):
                    plsc.store_scatter(out_v, [zvec, sidx[g]], vals[g])
                return (tuple(ix + feats for ix in gidx),
                        tuple(ix + 1 for ix in sidx))

            lax.fori_loop(0, bl, row_body, (gidx0, sidx0), unroll=2)
            make_out(bt_local, buf).start()
        for bt_local in range(max(bt_per_w - 2, 0), bt_per_w):
            make_out(bt_local, bt_local % 2).wait()

    return pl.kernel(
        body,
        out_type=jax.ShapeDtypeStruct((rows, feats), jnp.float32),
        mesh=mesh,
        scratch_types=[
            pltpu.VMEM((k,), jnp.int32),
            pltpu.VMEM((2, k), jnp.int32),
            pltpu.VMEM((2, bl, feats), jnp.float32),
            pltpu.VMEM((2, k, bl), jnp.float32),
            (pltpu.SemaphoreType.DMA,) * 4,
        ],
        compiler_params=pltpu.CompilerParams(needs_layout_passes=False),
    )(params, mi_flat)


def kernel(parameters, marginal_indices):
    rows, feats = parameters.shape
    m, t = marginal_indices.shape
    mi_flat = marginal_indices.reshape(-1).astype(jnp.int32)
    res = _sc_gather_t(parameters, mi_flat, bt_per_w=rows // (128 * NW))
    return (res.reshape(m, rows // 128, t, 128)
            .transpose(1, 3, 0, 2)
            .reshape(rows, m, t))


# speedup vs baseline: 1.8469x; 1.8469x over previous
import functools

import jax
import jax.numpy as jnp
from jax import lax
from jax.experimental import pallas as pl
from jax.experimental.pallas import tpu as pltpu
from jax.experimental.pallas import tpu_sc as plsc

NC = 2
NS = 16
L = 16
NW = NC * NS


@functools.partial(jax.jit, static_argnames=("bt_per_w",))
def _sc_gather_t(params, mi_flat, *, bt_per_w):
    rows, feats = params.shape
    k = mi_flat.shape[0]
    kv = k // L
    bl = 128
    mesh = plsc.VectorSubcoreMesh(
        core_axis_name="c", subcore_axis_name="s", num_cores=NC, num_subcores=NS
    )

    skew = bl + 1
    skrows = (k * skew + bl - 1) // bl + 1

    def body(p_hbm, mi_hbm, out_hbm, mi_v, idx_vs, in_vs, sk_v, out_vs, sems):
        sem_i0, sem_i1, sem_o0, sem_o1 = sems
        sem_i = (sem_i0, sem_i1)
        sem_o = (sem_o0, sem_o1)
        wid = lax.axis_index("s") * NC + lax.axis_index("c")
        pltpu.sync_copy(mi_hbm, mi_v)
        lane = lax.iota(jnp.int32, L)
        zvec = jnp.zeros((L,), jnp.int32)
        colbase = [(lane + g * L) * skew for g in range(kv)]
        rpbase = [lane + g * L for g in range(kv)]
        rbase = [(lane + g * L) // 2 * 256 + (lane + g * L) % 2
                 for g in range(kv)]

        def make_in(bt_local, buf):
            bt = (wid * bt_per_w + bt_local) * bl
            return pltpu.make_async_copy(
                p_hbm.at[pl.ds(bt, bl)], in_vs.at[buf], sem_i[buf])

        def make_out(bt_local, buf):
            return pltpu.make_async_copy(
                out_vs.at[buf], out_hbm.at[idx_vs.at[buf]], sem_o[buf])

        make_in(0, 0).start()
        for bt_local in range(bt_per_w):
            buf = bt_local % 2
            if bt_local + 1 < bt_per_w:
                make_in(bt_local + 1, 1 - buf).start()
            if bt_local >= 2:
                make_out(bt_local - 2, buf).wait()
            bt2 = (wid * bt_per_w + bt_local) * 2
            for g in range(kv):
                idx_vs[buf, pl.ds(g * L, L)] = rbase[g] + bt2
            make_in(bt_local, buf).wait()
            in_v = in_vs.at[buf]
            out_v = out_vs.at[buf]
            gidx0 = tuple(mi_v[pl.ds(g * L, L)] for g in range(kv))
            sidx0 = tuple(colbase[g] for g in range(kv))

            def row_body(b, carry):
                gidx, sidx = carry
                vals = [plsc.load_gather(in_v, [zvec, gidx[g]])
                        for g in range(kv)]
                for g in range(kv):
                    plsc.store_scatter(
                        sk_v, [sidx[g] >> 7, sidx[g] & (bl - 1)], vals[g])
                return (tuple(ix + feats for ix in gidx),
                        tuple(ix + 1 for ix in sidx))

            lax.fori_loop(0, bl, row_body, (gidx0, sidx0), unroll=2)

            def repack_body(f, _):
                fs = f * skew
                vals = [plsc.load_gather(
                    sk_v, [(rpbase[g] + fs) >> 7, (rpbase[g] + fs) & (bl - 1)])
                    for g in range(kv)]
                for g in range(kv):
                    out_v[f, pl.ds(g * L, L)] = vals[g]
                return 0

            lax.fori_loop(0, k, repack_body, 0, unroll=2)
            make_out(bt_local, buf).start()
        for bt_local in range(max(bt_per_w - 2, 0), bt_per_w):
            make_out(bt_local, bt_local % 2).wait()

    return pl.kernel(
        body,
        out_type=jax.ShapeDtypeStruct((rows, feats), jnp.float32),
        mesh=mesh,
        scratch_types=[
            pltpu.VMEM((k,), jnp.int32),
            pltpu.VMEM((2, k), jnp.int32),
            pltpu.VMEM((2, bl, feats), jnp.float32),
            pltpu.VMEM((skrows, bl), jnp.float32),
            pltpu.VMEM((2, k, bl), jnp.float32),
            (pltpu.SemaphoreType.DMA,) * 4,
        ],
        compiler_params=pltpu.CompilerParams(needs_layout_passes=False),
    )(params, mi_flat)


def kernel(parameters, marginal_indices):
    rows, feats = parameters.shape
    m, t = marginal_indices.shape
    mi_flat = marginal_indices.reshape(-1).astype(jnp.int32)
    res = _sc_gather_t(parameters, mi_flat, bt_per_w=rows // (128 * NW))
    return (res.reshape(m, rows // 128, t, 128)
            .transpose(1, 3, 0, 2)
            .reshape(rows, m, t))


# speedup vs baseline: 1.8507x; 1.0020x over previous
import functools

import jax
import jax.numpy as jnp
from jax import lax
from jax.experimental import pallas as pl
from jax.experimental.pallas import tpu as pltpu
from jax.experimental.pallas import tpu_sc as plsc

NC = 2
NS = 16
L = 16
NW = NC * NS


@functools.partial(jax.jit, static_argnames=("bt_per_w",))
def _sc_gather_t(params, mi_flat, *, bt_per_w):
    rows, feats = params.shape
    k = mi_flat.shape[0]
    kv = k // L
    bl = 128
    mesh = plsc.VectorSubcoreMesh(
        core_axis_name="c", subcore_axis_name="s", num_cores=NC, num_subcores=NS
    )

    skew = bl + 1
    skrows = (k * skew + bl - 1) // bl + 1

    def body(p_hbm, mi_hbm, out_hbm, mi_v, idx_vs, in_vs, sk_v, out_vs, sems):
        sem_i0, sem_i1, sem_o0, sem_o1 = sems
        sem_i = (sem_i0, sem_i1)
        sem_o = (sem_o0, sem_o1)
        wid = lax.axis_index("s") * NC + lax.axis_index("c")
        pltpu.sync_copy(mi_hbm, mi_v)
        lane = lax.iota(jnp.int32, L)
        zvec = jnp.zeros((L,), jnp.int32)
        fvec = [(lane + g * L) * 2 for g in range(kv // 2)] + \
               [(lane + g * L) * 2 + 1 for g in range(kv // 2)]
        colbase = [fvec[g] * skew for g in range(kv)]
        rpbase = [lane + g * L for g in range(kv)]
        rbase = [(lane + g * L) // 2 * 256 + (lane + g * L) % 2
                 for g in range(kv)]

        def make_in(bt_local, buf):
            bt = (wid * bt_per_w + bt_local) * bl
            return pltpu.make_async_copy(
                p_hbm.at[pl.ds(bt, bl)], in_vs.at[buf], sem_i[buf])

        def make_out(bt_local, buf):
            return pltpu.make_async_copy(
                out_vs.at[buf], out_hbm.at[idx_vs.at[buf]], sem_o[buf])

        make_in(0, 0).start()
        for bt_local in range(bt_per_w):
            buf = bt_local % 2
            if bt_local + 1 < bt_per_w:
                make_in(bt_local + 1, 1 - buf).start()
            if bt_local >= 2:
                make_out(bt_local - 2, buf).wait()
            bt2 = (wid * bt_per_w + bt_local) * 2
            for g in range(kv):
                idx_vs[buf, pl.ds(g * L, L)] = rbase[g] + bt2
            make_in(bt_local, buf).wait()
            in_v = in_vs.at[buf]
            out_v = out_vs.at[buf]
            gidx0 = tuple(plsc.load_gather(mi_v, [fvec[g]]) for g in range(kv))
            sidx0 = tuple(colbase[g] for g in range(kv))

            def row_body(b, carry):
                gidx, sidx = carry
                vals = [plsc.load_gather(in_v, [zvec, gidx[g]])
                        for g in range(kv)]
                for g in range(kv):
                    plsc.store_scatter(sk_v, [sidx[g]], vals[g])
                return (tuple(ix + feats for ix in gidx),
                        tuple(ix + 1 for ix in sidx))

            lax.fori_loop(0, bl, row_body, (gidx0, sidx0), unroll=2)

            def repack_body(f, _):
                fs = f * skew
                vals = [plsc.load_gather(sk_v, [rpbase[g] + fs])
                        for g in range(kv)]
                for g in range(kv):
                    out_v[f, pl.ds(g * L, L)] = vals[g]
                return 0

            lax.fori_loop(0, k, repack_body, 0, unroll=2)
            make_out(bt_local, buf).start()
        for bt_local in range(max(bt_per_w - 2, 0), bt_per_w):
            make_out(bt_local, bt_local % 2).wait()

    return pl.kernel(
        body,
        out_type=jax.ShapeDtypeStruct((rows, feats), jnp.float32),
        mesh=mesh,
        scratch_types=[
            pltpu.VMEM((k,), jnp.int32),
            pltpu.VMEM((2, k), jnp.int32),
            pltpu.VMEM((2, bl, feats), jnp.float32),
            pltpu.VMEM((skrows * bl,), jnp.float32),
            pltpu.VMEM((2, k, bl), jnp.float32),
            (pltpu.SemaphoreType.DMA,) * 4,
        ],
        compiler_params=pltpu.CompilerParams(needs_layout_passes=False),
    )(params, mi_flat)


def kernel(parameters, marginal_indices):
    rows, feats = parameters.shape
    m, t = marginal_indices.shape
    mi_flat = marginal_indices.reshape(-1).astype(jnp.int32)
    res = _sc_gather_t(parameters, mi_flat, bt_per_w=rows // (128 * NW))
    return (res.reshape(m, rows // 128, t, 128)
            .transpose(1, 3, 0, 2)
            .reshape(rows, m, t))
